# baseline (speedup 1.0000x reference)
import jax
import jax.numpy as jnp
from jax import lax
from jax.experimental import pallas as pl
from jax.experimental.pallas import tpu as pltpu

_INTEGRAL = 1.0
_BB = 4096


def _cnf_kernel(t_ref, zt_ref, w1t_ref, b1c_ref, w2t_ref, b2c_ref,
                ft_ref, dlt_ref):
    t = t_ref[0]
    zt = zt_ref[...]
    w1t = w1t_ref[...]
    w1zt = w1t[:, 1:]
    w2t = w2t_ref[...]

    a = -_INTEGRAL * t
    tm = jnp.full((1, 1), t, dtype=jnp.float32)
    b = a * lax.rsqrt(1.0 - jnp.exp(-(_INTEGRAL * tm * tm)))

    pre = jnp.dot(w1zt, zt, preferred_element_type=jnp.float32)
    pre = pre + (t * w1t[:, 0:1] + b1c_ref[...])
    h = jnp.tanh(pre)
    mlp = jnp.dot(w2t, h, preferred_element_type=jnp.float32) + b2c_ref[...]
    ft_ref[...] = a * zt - b * mlp

    g = jnp.dot(w1zt, w2t, preferred_element_type=jnp.float32)
    hh = g.shape[0]
    rows = lax.broadcasted_iota(jnp.int32, (hh, hh), 0)
    cols = lax.broadcasted_iota(jnp.int32, (hh, hh), 1)
    c_col = jnp.sum(jnp.where(rows == cols, g, 0.0), axis=1, keepdims=True)
    c0 = jnp.sum(c_col, axis=0, keepdims=True)
    tr_neg = jnp.sum((h * h) * (b * c_col), axis=0, keepdims=True)
    dlt_ref[...] = (b * c0 - a * jnp.float32(zt.shape[0])) - tr_neg


def kernel(t, z, W1, b1, W2, b2):
    B, D = z.shape
    H = W2.shape[0]

    zt = z.T
    w1t = W1.T
    w2t = W2.T
    b1c = b1.reshape(H, 1)
    b2c = b2.reshape(D, 1)

    grid = (B // _BB,)
    ft, dlt = pl.pallas_call(
        _cnf_kernel,
        grid=grid,
        in_specs=[
            pl.BlockSpec(memory_space=pltpu.SMEM),
            pl.BlockSpec((D, _BB), lambda i: (0, i)),
            pl.BlockSpec((H, D + 1), lambda i: (0, 0)),
            pl.BlockSpec((H, 1), lambda i: (0, 0)),
            pl.BlockSpec((D, H), lambda i: (0, 0)),
            pl.BlockSpec((D, 1), lambda i: (0, 0)),
        ],
        out_specs=[
            pl.BlockSpec((D, _BB), lambda i: (0, i)),
            pl.BlockSpec((1, _BB), lambda i: (0, i)),
        ],
        out_shape=[
            jax.ShapeDtypeStruct((D, B), jnp.float32),
            jax.ShapeDtypeStruct((1, B), jnp.float32),
        ],
        compiler_params=pltpu.CompilerParams(
            dimension_semantics=("parallel",),
        ),
        name="cnf_trace_fused_t",
    )(t, zt, w1t, b1c, w2t, b2c)
    return ft.T, dlt.T


# speedup vs baseline: 1.0046x; 1.0046x over previous
import jax
import jax.numpy as jnp
from jax import lax
from jax.experimental import pallas as pl
from jax.experimental.pallas import tpu as pltpu

_INTEGRAL = 1.0
_BB = 2048


def _cnf_kernel(t_ref, zt_ref, w1t_ref, b1c_ref, w2t_ref, b2c_ref,
                ft_ref, dlt_ref):
    t = t_ref[0]
    zt = zt_ref[...]
    w1t = w1t_ref[...]
    w1zt = w1t[:, 1:]
    w2t = w2t_ref[...]

    a = -_INTEGRAL * t
    tm = jnp.full((1, 1), t, dtype=jnp.float32)
    b = a * lax.rsqrt(1.0 - jnp.exp(-(_INTEGRAL * tm * tm)))

    pre = jnp.dot(w1zt, zt, preferred_element_type=jnp.float32)
    pre = pre + (t * w1t[:, 0:1] + b1c_ref[...])
    h = jnp.tanh(pre)
    mlp = jnp.dot(w2t, h, preferred_element_type=jnp.float32) + b2c_ref[...]
    ft_ref[...] = a * zt - b * mlp

    g = jnp.dot(w1zt, w2t, preferred_element_type=jnp.float32)
    hh = g.shape[0]
    rows = lax.broadcasted_iota(jnp.int32, (hh, hh), 0)
    cols = lax.broadcasted_iota(jnp.int32, (hh, hh), 1)
    c_row = jnp.sum(jnp.where(rows == cols, g, 0.0), axis=0, keepdims=True)
    c0 = jnp.sum(c_row, axis=1, keepdims=True)
    tr_neg = jnp.dot(b * c_row, h * h, preferred_element_type=jnp.float32)
    dlt_ref[...] = (b * c0 - a * jnp.float32(zt.shape[0])) - tr_neg


def kernel(t, z, W1, b1, W2, b2):
    B, D = z.shape
    H = W2.shape[0]

    zt = z.T
    w1t = W1.T
    w2t = W2.T
    b1c = b1.reshape(H, 1)
    b2c = b2.reshape(D, 1)

    grid = (B // _BB,)
    ft, dlt = pl.pallas_call(
        _cnf_kernel,
        grid=grid,
        in_specs=[
            pl.BlockSpec(memory_space=pltpu.SMEM),
            pl.BlockSpec((D, _BB), lambda i: (0, i)),
            pl.BlockSpec((H, D + 1), lambda i: (0, 0)),
            pl.BlockSpec((H, 1), lambda i: (0, 0)),
            pl.BlockSpec((D, H), lambda i: (0, 0)),
            pl.BlockSpec((D, 1), lambda i: (0, 0)),
        ],
        out_specs=[
            pl.BlockSpec((D, _BB), lambda i: (0, i)),
            pl.BlockSpec((1, _BB), lambda i: (0, i)),
        ],
        out_shape=[
            jax.ShapeDtypeStruct((D, B), jnp.float32),
            jax.ShapeDtypeStruct((1, B), jnp.float32),
        ],
        compiler_params=pltpu.CompilerParams(
            dimension_semantics=("parallel",),
        ),
        name="cnf_trace_fused_t",
    )(t, zt, w1t, b1c, w2t, b2c)
    return ft.T, dlt.T
